# baseline (speedup 1.0000x reference)
import jax
import jax.numpy as jnp
from jax.experimental import pallas as pl

_NUM_CLASSES = 1000
_BATCH = 16384
_BLOCK_ROWS = 512


def _onehot_body(x_ref, o_ref):
    ids = x_ref[0, 0, :].astype(jnp.int32)
    cols = jax.lax.broadcasted_iota(jnp.int32, (_BLOCK_ROWS, _NUM_CLASSES), 1)
    o_ref[...] = (cols == ids[:, None]).astype(jnp.float32)


def kernel(x1):
    x = x1.astype(jnp.int32).reshape(_BATCH // _BLOCK_ROWS, 1, _BLOCK_ROWS)
    grid = (_BATCH // _BLOCK_ROWS,)
    return pl.pallas_call(
        _onehot_body,
        grid=grid,
        in_specs=[pl.BlockSpec((1, 1, _BLOCK_ROWS), lambda i: (i, 0, 0))],
        out_specs=pl.BlockSpec((_BLOCK_ROWS, _NUM_CLASSES), lambda i: (i, 0)),
        out_shape=jax.ShapeDtypeStruct((_BATCH, _NUM_CLASSES), jnp.float32),
    )(x)
